# baseline (speedup 1.0000x reference)
import functools

import jax
import jax.numpy as jnp
from jax import lax
from jax.experimental import pallas as pl
from jax.experimental.pallas import tpu as pltpu
from jax.experimental.pallas import tpu_sc as plsc

B, N, D, C, K = 8, 256, 25, 32, 20
DC = D * C
ROWS = B * N
LANES = 16
NC, NS = 2, 16
NW = NC * NS
RPW = ROWS // NW
G = 2
NCH = RPW // G
IPC = G * K
NB = 3


def _fm_body(x_hbm, idx_hbm, w1t_hbm, out_hbm, idx_v, w1t_v, rows_v, out_v,
             gsems, osems):
    wid = lax.axis_index("s") * NC + lax.axis_index("c")

    pltpu.sync_copy(idx_hbm.at[pl.ds(wid * NCH, NCH)], idx_v)
    pltpu.sync_copy(w1t_hbm, w1t_v)

    def wraw(k, h):
        p = 2 * k + h
        return w1t_v[p // 8, pl.ds((p % 8) * LANES, LANES)]

    invs = []
    for h in range(2):
        s = jnp.zeros((LANES,), jnp.float32)
        for k in range(K):
            a = wraw(k, h)
            s = s + a * a
        invs.append(1.0 / s)

    def start_gather(j, b):
        pltpu.async_copy(x_hbm.at[idx_v.at[j, pl.ds(0, IPC)]], rows_v.at[b],
                         gsems.at[b])

    def wait_gather(b):
        pltpu.make_async_copy(x_hbm.at[idx_v.at[0, pl.ds(0, IPC)]],
                              rows_v.at[b], gsems.at[b]).wait()

    def wait_out(b):
        pltpu.make_async_copy(out_v.at[b], out_hbm.at[pl.ds(0, G)],
                              osems.at[b]).wait()

    for b in range(NB):
        start_gather(b, b)

    def emit_chunk(j, b, first):
        wait_gather(b)
        if first:
            @pl.when(j >= NB)
            def _():
                wait_out(b)
        else:
            wait_out(b)

        for h in range(2):
            wn = [wraw(k, h) * wraw(k, h) * invs[h] for k in range(K)]

            def col(d, c2, _wn=wn, _h=h):
                sl = pl.ds(_h * LANES, LANES)
                lo, hi = _h * LANES, (_h + 1) * LANES

                def half(r, _d):
                    return rows_v[b, r, _d].astype(jnp.float32)[lo:hi]

                for g in range(G):
                    acc0 = half(g * K, d) * _wn[0]
                    acc1 = half(g * K + 1, d) * _wn[1]
                    for k in range(2, K, 2):
                        acc0 = acc0 + half(g * K + k, d) * _wn[k]
                        acc1 = acc1 + half(g * K + k + 1, d) * _wn[k + 1]
                    out_v[b, g, d, sl] = acc0 + acc1
                return c2

            lax.fori_loop(0, D, col, 0)

        pltpu.async_copy(out_v.at[b],
                         out_hbm.at[pl.ds(wid * RPW + j * G, G)],
                         osems.at[b])

        @pl.when(j + NB < NCH)
        def _():
            start_gather(j + NB, b)

    NFULL = (NCH // NB) * NB

    def chunk_trip(j2, carry):
        for b in range(NB):
            emit_chunk(j2 * NB + b, b, first=True)
        return carry

    lax.fori_loop(0, NFULL // NB, chunk_trip, 0)
    for jj in range(NFULL, NCH):
        emit_chunk(jj, jj % NB, first=False)
    for b in range(NB):
        wait_out(b)


@jax.jit
def _fm_call(x3, idx, w1t):
    mesh = plsc.VectorSubcoreMesh(core_axis_name="c", subcore_axis_name="s")
    run = functools.partial(
        pl.kernel,
        mesh=mesh,
        out_type=jax.ShapeDtypeStruct((ROWS, D, C), jnp.float32),
        scratch_types=[
            pltpu.VMEM((NCH, 128), jnp.int32),
            pltpu.VMEM((8, 128), jnp.float32),
            pltpu.VMEM((NB, IPC, D, C), jnp.bfloat16),
            pltpu.VMEM((NB, G, D, C), jnp.float32),
            pltpu.SemaphoreType.DMA((NB,)),
            pltpu.SemaphoreType.DMA((NB,)),
        ],
        compiler_params=pltpu.CompilerParams(use_tc_tiling_on_sc=False),
    )(_fm_body)
    return run(x3, idx, w1t)


def kernel(x, knn_matrix, w1, conv_w, conv_b):
    del conv_w
    x3 = x.astype(jnp.bfloat16).reshape(ROWS, D, C)
    flat_idx = (knn_matrix.astype(jnp.int32)
                + (jnp.arange(B, dtype=jnp.int32) * N).reshape(B, 1, 1))
    idx = jnp.pad(flat_idx.reshape(NW * NCH, IPC),
                  ((0, 0), (0, 128 - IPC)))
    w1t = jnp.pad(w1.T.reshape(-1), (0, 8 * 128 - K * C)).reshape(8, 128)
    out = _fm_call(x3, idx, w1t)
    return out.reshape(B, N, D, C)


# speedup vs baseline: 1.2577x; 1.2577x over previous
import functools

import jax
import jax.numpy as jnp
from jax import lax
from jax.experimental import pallas as pl
from jax.experimental.pallas import tpu as pltpu
from jax.experimental.pallas import tpu_sc as plsc

B, N, D, C, K = 8, 256, 25, 32, 20
DC = D * C
ROWS = B * N
LANES = 16
NC, NS = 2, 16
NW = NC * NS
RPW = ROWS // NW
G = 2
NCH = RPW // G
IPC = G * K
NB = 3


def _fm_body(x_hbm, idx_hbm, w1t_hbm, out_hbm, idx_v, w1t_v, rows_v, out_v,
             gsems, osems):
    wid = lax.axis_index("s") * NC + lax.axis_index("c")

    pltpu.sync_copy(idx_hbm.at[pl.ds(wid * NCH, NCH)], idx_v)
    pltpu.sync_copy(w1t_hbm, w1t_v)

    def wraw(k, h):
        p = 2 * k + h
        return w1t_v[p // 8, pl.ds((p % 8) * LANES, LANES)]

    invs = []
    for h in range(2):
        s = jnp.zeros((LANES,), jnp.float32)
        for k in range(K):
            a = wraw(k, h)
            s = s + a * a
        invs.append(1.0 / s)

    def start_gather(j, b):
        pltpu.async_copy(x_hbm.at[idx_v.at[j, pl.ds(0, IPC)]], rows_v.at[b],
                         gsems.at[b])

    def wait_gather(b):
        pltpu.make_async_copy(x_hbm.at[idx_v.at[0, pl.ds(0, IPC)]],
                              rows_v.at[b], gsems.at[b]).wait()

    def wait_out(b):
        pltpu.make_async_copy(out_v.at[b], out_hbm.at[pl.ds(0, G)],
                              osems.at[b]).wait()

    for b in range(NB):
        start_gather(b, b)

    def emit_chunk(j, b, first):
        wait_gather(b)
        if first:
            @pl.when(j >= NB)
            def _():
                wait_out(b)
        else:
            wait_out(b)

        for h in range(2):
            wn = [wraw(k, h) * wraw(k, h) * invs[h] for k in range(K)]

            def col(d, c2, _wn=wn, _h=h):
                sl = pl.ds(_h * LANES, LANES)
                for g in range(G):
                    acc0 = rows_v[b, g * K, d, sl] * _wn[0]
                    acc1 = rows_v[b, g * K + 1, d, sl] * _wn[1]
                    for k in range(2, K, 2):
                        acc0 = acc0 + rows_v[b, g * K + k, d, sl] * _wn[k]
                        acc1 = (acc1
                                + rows_v[b, g * K + k + 1, d, sl]
                                * _wn[k + 1])
                    out_v[b, g, d, sl] = acc0 + acc1
                return c2

            lax.fori_loop(0, D, col, 0)

        pltpu.async_copy(out_v.at[b],
                         out_hbm.at[pl.ds(wid * RPW + j * G, G)],
                         osems.at[b])

        @pl.when(j + NB < NCH)
        def _():
            start_gather(j + NB, b)

    NFULL = (NCH // NB) * NB

    def chunk_trip(j2, carry):
        for b in range(NB):
            emit_chunk(j2 * NB + b, b, first=True)
        return carry

    lax.fori_loop(0, NFULL // NB, chunk_trip, 0)
    for jj in range(NFULL, NCH):
        emit_chunk(jj, jj % NB, first=False)
    for b in range(NB):
        wait_out(b)


@jax.jit
def _fm_call(x3, idx, w1t):
    mesh = plsc.VectorSubcoreMesh(core_axis_name="c", subcore_axis_name="s")
    run = functools.partial(
        pl.kernel,
        mesh=mesh,
        out_type=jax.ShapeDtypeStruct((ROWS, D, C), jnp.float32),
        scratch_types=[
            pltpu.VMEM((NCH, 128), jnp.int32),
            pltpu.VMEM((8, 128), jnp.float32),
            pltpu.VMEM((NB, IPC, D, C), jnp.float32),
            pltpu.VMEM((NB, G, D, C), jnp.float32),
            pltpu.SemaphoreType.DMA((NB,)),
            pltpu.SemaphoreType.DMA((NB,)),
        ],
        compiler_params=pltpu.CompilerParams(use_tc_tiling_on_sc=False),
    )(_fm_body)
    return run(x3, idx, w1t)


def kernel(x, knn_matrix, w1, conv_w, conv_b):
    del conv_w
    x3 = x.reshape(ROWS, D, C)
    flat_idx = (knn_matrix.astype(jnp.int32)
                + (jnp.arange(B, dtype=jnp.int32) * N).reshape(B, 1, 1))
    idx = jnp.pad(flat_idx.reshape(NW * NCH, IPC),
                  ((0, 0), (0, 128 - IPC)))
    w1t = jnp.pad(w1.T.reshape(-1), (0, 8 * 128 - K * C)).reshape(8, 128)
    out = _fm_call(x3, idx, w1t)
    return out.reshape(B, N, D, C)
